# Initial kernel scaffold; baseline (speedup 1.0000x reference)
import jax
import jax.numpy as jnp
from jax.experimental import pallas as pl


def kernel(out_boxes, tgt_boxes):
    raise NotImplementedError("write your pallas kernel here")


# speedup vs baseline: 2.1403x; 2.1403x over previous
import jax
import jax.numpy as jnp
from jax.experimental import pallas as pl
from jax.experimental.pallas import tpu as pltpu

_BIG = 1e30
_LANE = 128


def _matcher_body(pred_ref, tgt_ref, out_ref, cost_ref,
                  u_ref, v_ref, r4c_ref, c4r_ref, c4rcol_ref,
                  sr_ref, rowminv_ref, sc_ref, spc_ref, path_ref):
    G = cost_ref.shape[1]
    M = cost_ref.shape[0]

    tx1 = tgt_ref[0, :, 0:1]
    ty1 = tgt_ref[0, :, 1:2]
    tx2 = tgt_ref[0, :, 2:3]
    ty2 = tgt_ref[0, :, 3:4]
    area_t = (tx2 - tx1) * (ty2 - ty1)
    for g in range(G):
        px1 = pred_ref[0, 0, g, :]
        py1 = pred_ref[0, 1, g, :]
        px2 = pred_ref[0, 2, g, :]
        py2 = pred_ref[0, 3, g, :]
        c_l1 = ((jnp.abs(px1 - tx1) + jnp.abs(py1 - ty1))
                + jnp.abs(px2 - tx2)) + jnp.abs(py2 - ty2)
        area_p = (px2 - px1) * (py2 - py1)
        ltx = jnp.maximum(px1, tx1)
        lty = jnp.maximum(py1, ty1)
        rbx = jnp.minimum(px2, tx2)
        rby = jnp.minimum(py2, ty2)
        iw = jnp.maximum(rbx - ltx, 0.0)
        ih = jnp.maximum(rby - lty, 0.0)
        inter = iw * ih
        union = (area_p + area_t) - inter
        iou = inter / (union + 1e-8)
        elx = jnp.minimum(px1, tx1)
        ely = jnp.minimum(py1, ty1)
        erx = jnp.maximum(px2, tx2)
        ery = jnp.maximum(py2, ty2)
        ew = jnp.maximum(erx - elx, 0.0)
        eh = jnp.maximum(ery - ely, 0.0)
        earea = ew * eh
        giou = iou - (earea - union) / (earea + 1e-8)
        cost_ref[:, g, :] = c_l1 - giou

    riota = jax.lax.broadcasted_iota(jnp.int32, (1, M), 1)
    citer = jax.lax.broadcasted_iota(jnp.int32, (M, 1), 0)
    fiota = (jax.lax.broadcasted_iota(jnp.int32, (G, _LANE), 0) * _LANE
             + jax.lax.broadcasted_iota(jnp.int32, (G, _LANE), 1))

    u_ref[...] = jnp.zeros((1, M), jnp.float32)
    v_ref[...] = jnp.zeros((G, _LANE), jnp.float32)
    r4c_ref[...] = jnp.full((G, _LANE), -1, jnp.int32)
    c4r_ref[...] = jnp.full((1, M), -1, jnp.int32)
    c4rcol_ref[...] = jnp.full((M, 1), -1, jnp.int32)

    def outer(i, _):
        sr_ref[...] = jnp.zeros((1, M), jnp.int32)
        rowminv_ref[...] = jnp.zeros((1, M), jnp.float32)
        sc_ref[...] = jnp.zeros((G, _LANE), jnp.int32)
        spc_ref[...] = jnp.full((G, _LANE), _BIG, jnp.float32)
        path_ref[...] = jnp.full((G, _LANE), -1, jnp.int32)

        def sp_cond(s):
            return s[2] < 0

        def sp_body(s):
            cur, minv, _sink = s
            sr_ref[...] = jnp.where(riota == cur, 1, sr_ref[...])
            ucur = jnp.sum(jnp.where(riota == cur, u_ref[...], 0.0))
            crow = cost_ref[cur]
            red = ((minv + crow) - ucur) - v_ref[...]
            sc = sc_ref[...]
            spc = spc_ref[...]
            better = (sc == 0) & (red < spc)
            spc = jnp.where(better, red, spc)
            spc_ref[...] = spc
            path_ref[...] = jnp.where(better, cur, path_ref[...])
            masked = jnp.where(sc != 0, _BIG, spc)
            minv_new = jnp.min(masked)
            j = jnp.min(jnp.where(masked == minv_new, fiota, jnp.int32(1 << 30)))
            sc_ref[...] = jnp.where(fiota == j, 1, sc)
            r4cj = jnp.sum(jnp.where(fiota == j, r4c_ref[...], 0))
            unassigned = r4cj < 0
            sink = jnp.where(unassigned, j, jnp.int32(-1))
            cur = jnp.where(unassigned, cur, r4cj)
            rowminv_ref[...] = jnp.where(riota == r4cj, minv_new, rowminv_ref[...])
            return (cur, minv_new, sink)

        _, minv, sink = jax.lax.while_loop(
            sp_cond, sp_body, (i, jnp.float32(0.0), jnp.int32(-1)))

        u = u_ref[...]
        u = jnp.where(riota == i, u + minv, u)
        rmask = (sr_ref[...] != 0) & (riota != i)
        u_ref[...] = jnp.where(rmask, (u + minv) - rowminv_ref[...], u)
        sc = sc_ref[...] != 0
        v = v_ref[...]
        v_ref[...] = jnp.where(sc, (v + spc_ref[...]) - minv, v)

        def aug_cond(s):
            return s[1] == 0

        def aug_body(s):
            j, _ = s
            ii = jnp.sum(jnp.where(fiota == j, path_ref[...], 0))
            r4c_ref[...] = jnp.where(fiota == j, ii, r4c_ref[...])
            nj = jnp.sum(jnp.where(riota == ii, c4r_ref[...], 0))
            c4r_ref[...] = jnp.where(riota == ii, j, c4r_ref[...])
            c4rcol_ref[...] = jnp.where(citer == ii, j, c4rcol_ref[...])
            return (nj, jnp.where(ii == i, jnp.int32(1), jnp.int32(0)))

        jax.lax.while_loop(aug_cond, aug_body, (sink, jnp.int32(0)))
        return 0

    jax.lax.fori_loop(0, M, outer, 0)

    c4r = c4r_ref[...]
    c4rcol = c4rcol_ref[...]
    less = c4r < c4rcol
    rank = jnp.sum(less.astype(jnp.int32), axis=1, keepdims=True)
    onehot = rank == riota
    pred_sorted = jnp.sum(jnp.where(onehot, c4rcol, 0), axis=0, keepdims=True)
    tgt_sorted = jnp.sum(jnp.where(onehot, citer, 0), axis=0, keepdims=True)
    out_ref[0, 0:1, :] = pred_sorted
    out_ref[0, 1:2, :] = tgt_sorted


def kernel(out_boxes, tgt_boxes):
    B, N, _ = out_boxes.shape
    M = tgt_boxes.shape[1]
    G = N // _LANE
    pred = jnp.transpose(out_boxes, (0, 2, 1)).reshape(B, 4, G, _LANE)
    pred = jnp.pad(pred, ((0, 0), (0, 4), (0, 0), (0, 0)))
    tgt = jnp.pad(tgt_boxes, ((0, 0), (0, 0), (0, 4)))
    out = pl.pallas_call(
        _matcher_body,
        grid=(B,),
        in_specs=[pl.BlockSpec((1, 8, G, _LANE), lambda b: (b, 0, 0, 0)),
                  pl.BlockSpec((1, M, 8), lambda b: (b, 0, 0))],
        out_specs=pl.BlockSpec((1, 8, M), lambda b: (b, 0, 0)),
        out_shape=jax.ShapeDtypeStruct((B, 8, M), jnp.int32),
        scratch_shapes=[pltpu.VMEM((M, G, _LANE), jnp.float32),
                        pltpu.VMEM((1, M), jnp.float32),
                        pltpu.VMEM((G, _LANE), jnp.float32),
                        pltpu.VMEM((G, _LANE), jnp.int32),
                        pltpu.VMEM((1, M), jnp.int32),
                        pltpu.VMEM((M, 1), jnp.int32),
                        pltpu.VMEM((1, M), jnp.int32),
                        pltpu.VMEM((1, M), jnp.float32),
                        pltpu.VMEM((G, _LANE), jnp.int32),
                        pltpu.VMEM((G, _LANE), jnp.float32),
                        pltpu.VMEM((G, _LANE), jnp.int32)],
        compiler_params=pltpu.CompilerParams(
            dimension_semantics=("parallel",)),
    )(pred, tgt)
    return out[:, :2, :]


# speedup vs baseline: 3.0204x; 1.4112x over previous
import jax
import jax.numpy as jnp
from jax.experimental import pallas as pl
from jax.experimental.pallas import tpu as pltpu

_BIG = 1e30
_LANE = 128


def _matcher_body(pred_ref, tgt_ref, out_ref, cost_ref,
                  u_ref, v_ref, r4c_ref, c4r_ref, c4rcol_ref,
                  sr_ref, rowminv_ref, sc_ref, spc_ref, path_ref, minv_ref):
    G = cost_ref.shape[1]
    M = cost_ref.shape[0]

    tx1 = tgt_ref[0, :, 0:1]
    ty1 = tgt_ref[0, :, 1:2]
    tx2 = tgt_ref[0, :, 2:3]
    ty2 = tgt_ref[0, :, 3:4]
    area_t = (tx2 - tx1) * (ty2 - ty1)
    for g in range(G):
        px1 = pred_ref[0, 0, g, :]
        py1 = pred_ref[0, 1, g, :]
        px2 = pred_ref[0, 2, g, :]
        py2 = pred_ref[0, 3, g, :]
        c_l1 = ((jnp.abs(px1 - tx1) + jnp.abs(py1 - ty1))
                + jnp.abs(px2 - tx2)) + jnp.abs(py2 - ty2)
        area_p = (px2 - px1) * (py2 - py1)
        ltx = jnp.maximum(px1, tx1)
        lty = jnp.maximum(py1, ty1)
        rbx = jnp.minimum(px2, tx2)
        rby = jnp.minimum(py2, ty2)
        iw = jnp.maximum(rbx - ltx, 0.0)
        ih = jnp.maximum(rby - lty, 0.0)
        inter = iw * ih
        union = (area_p + area_t) - inter
        iou = inter / (union + 1e-8)
        elx = jnp.minimum(px1, tx1)
        ely = jnp.minimum(py1, ty1)
        erx = jnp.maximum(px2, tx2)
        ery = jnp.maximum(py2, ty2)
        ew = jnp.maximum(erx - elx, 0.0)
        eh = jnp.maximum(ery - ely, 0.0)
        earea = ew * eh
        giou = iou - (earea - union) / (earea + 1e-8)
        cost_ref[:, g, :] = c_l1 - giou

    riota = jax.lax.broadcasted_iota(jnp.int32, (1, M), 1)
    citer = jax.lax.broadcasted_iota(jnp.int32, (M, 1), 0)
    fiota = (jax.lax.broadcasted_iota(jnp.int32, (G, _LANE), 0) * _LANE
             + jax.lax.broadcasted_iota(jnp.int32, (G, _LANE), 1))

    u_ref[...] = jnp.zeros((1, M), jnp.float32)
    v_ref[...] = jnp.zeros((G, _LANE), jnp.float32)
    r4c_ref[...] = jnp.full((G, _LANE), -1, jnp.int32)
    c4r_ref[...] = jnp.full((1, M), -1, jnp.int32)
    c4rcol_ref[...] = jnp.full((M, 1), -1, jnp.int32)

    def outer(i, _):
        sr_ref[...] = jnp.zeros((1, M), jnp.int32)
        rowminv_ref[...] = jnp.zeros((1, M), jnp.float32)
        sc_ref[...] = jnp.zeros((G, _LANE), jnp.int32)
        spc_ref[...] = jnp.full((G, _LANE), _BIG, jnp.float32)
        path_ref[...] = jnp.zeros((G, _LANE), jnp.int32)
        minv_ref[...] = jnp.zeros((1, 1), jnp.float32)

        def sp_cond(s):
            return s[1] < 0

        def sp_body(s):
            cur, _sink = s
            minv = minv_ref[...]
            u = u_ref[...]
            sr_ref[...] = jnp.where(riota == cur, 1, sr_ref[...])
            curmask = riota == cur
            ucur = jnp.sum(jnp.where(curmask, u, 0.0), axis=1, keepdims=True)
            c4rcur = jnp.sum(jnp.where(curmask, c4r_ref[...], 0),
                             axis=1, keepdims=True)
            pathpk = (cur + 1) * 4096 + (c4rcur + 1)
            crow = cost_ref[cur]
            red = ((minv + crow) - ucur) - v_ref[...]
            sc = sc_ref[...]
            spc = spc_ref[...]
            better = (sc == 0) & (red < spc)
            spc = jnp.where(better, red, spc)
            spc_ref[...] = spc
            path_ref[...] = jnp.where(better, pathpk, path_ref[...])
            masked = jnp.where(sc != 0, _BIG, spc)
            minv_new = jnp.min(jnp.min(masked, axis=1, keepdims=True),
                               axis=0, keepdims=True)
            minv_ref[...] = minv_new
            key = jnp.where(masked == minv_new,
                            fiota * 256 + (r4c_ref[...] + 1), jnp.int32(1 << 30))
            kmin = jnp.min(key)
            j = kmin >> 8
            r4cj = (kmin & 255) - 1
            sc_ref[...] = jnp.where(fiota == j, 1, sc)
            unassigned = r4cj < 0
            sink = jnp.where(unassigned, j, jnp.int32(-1))
            cur = jnp.where(unassigned, cur, r4cj)
            rowminv_ref[...] = jnp.where(riota == r4cj, minv_new, rowminv_ref[...])
            return (cur, sink)

        _, sink = jax.lax.while_loop(sp_cond, sp_body, (i, jnp.int32(-1)))

        minv = minv_ref[...]
        u = u_ref[...]
        u = jnp.where(riota == i, u + minv, u)
        rmask = (sr_ref[...] != 0) & (riota != i)
        u_ref[...] = jnp.where(rmask, (u + minv) - rowminv_ref[...], u)
        sc = sc_ref[...] != 0
        v = v_ref[...]
        v_ref[...] = jnp.where(sc, (v + spc_ref[...]) - minv, v)

        def aug_cond(s):
            return s[1] == 0

        def aug_body(s):
            j, _ = s
            pk = jnp.sum(jnp.where(fiota == j, path_ref[...], 0))
            ii = (pk >> 12) - 1
            nj = (pk & 4095) - 1
            r4c_ref[...] = jnp.where(fiota == j, ii, r4c_ref[...])
            c4r_ref[...] = jnp.where(riota == ii, j, c4r_ref[...])
            c4rcol_ref[...] = jnp.where(citer == ii, j, c4rcol_ref[...])
            return (nj, jnp.where(ii == i, jnp.int32(1), jnp.int32(0)))

        jax.lax.while_loop(aug_cond, aug_body, (sink, jnp.int32(0)))
        return 0

    jax.lax.fori_loop(0, M, outer, 0)

    c4r = c4r_ref[...]
    c4rcol = c4rcol_ref[...]
    less = c4r < c4rcol
    rank = jnp.sum(less.astype(jnp.int32), axis=1, keepdims=True)
    onehot = rank == riota
    pred_sorted = jnp.sum(jnp.where(onehot, c4rcol, 0), axis=0, keepdims=True)
    tgt_sorted = jnp.sum(jnp.where(onehot, citer, 0), axis=0, keepdims=True)
    out_ref[0, 0:1, :] = pred_sorted
    out_ref[0, 1:2, :] = tgt_sorted


def kernel(out_boxes, tgt_boxes):
    B, N, _ = out_boxes.shape
    M = tgt_boxes.shape[1]
    G = N // _LANE
    pred = jnp.transpose(out_boxes, (0, 2, 1)).reshape(B, 4, G, _LANE)
    pred = jnp.pad(pred, ((0, 0), (0, 4), (0, 0), (0, 0)))
    tgt = jnp.pad(tgt_boxes, ((0, 0), (0, 0), (0, 4)))
    out = pl.pallas_call(
        _matcher_body,
        grid=(B,),
        in_specs=[pl.BlockSpec((1, 8, G, _LANE), lambda b: (b, 0, 0, 0)),
                  pl.BlockSpec((1, M, 8), lambda b: (b, 0, 0))],
        out_specs=pl.BlockSpec((1, 8, M), lambda b: (b, 0, 0)),
        out_shape=jax.ShapeDtypeStruct((B, 8, M), jnp.int32),
        scratch_shapes=[pltpu.VMEM((M, G, _LANE), jnp.float32),
                        pltpu.VMEM((1, M), jnp.float32),
                        pltpu.VMEM((G, _LANE), jnp.float32),
                        pltpu.VMEM((G, _LANE), jnp.int32),
                        pltpu.VMEM((1, M), jnp.int32),
                        pltpu.VMEM((M, 1), jnp.int32),
                        pltpu.VMEM((1, M), jnp.int32),
                        pltpu.VMEM((1, M), jnp.float32),
                        pltpu.VMEM((G, _LANE), jnp.int32),
                        pltpu.VMEM((G, _LANE), jnp.float32),
                        pltpu.VMEM((G, _LANE), jnp.int32),
                        pltpu.VMEM((1, 1), jnp.float32)],
        compiler_params=pltpu.CompilerParams(
            dimension_semantics=("parallel",)),
    )(pred, tgt)
    return out[:, :2, :]


# speedup vs baseline: 4.0662x; 1.3462x over previous
import jax
import jax.numpy as jnp
from jax.experimental import pallas as pl
from jax.experimental.pallas import tpu as pltpu

_BIG = 1e30
_LANE = 128
_P = 4


def _matcher_body(pred_ref, tgt_ref, out_ref, cost_ref,
                  u_ref, v_ref, r4c_ref, c4r_ref, c4rcol_ref,
                  sr_ref, rowminv_ref, sc_ref, spc_ref, path_ref, minv_ref):
    P = cost_ref.shape[0]
    G = cost_ref.shape[2]
    M = cost_ref.shape[1]

    for p in range(P):
        tx1 = tgt_ref[p, :, 0:1]
        ty1 = tgt_ref[p, :, 1:2]
        tx2 = tgt_ref[p, :, 2:3]
        ty2 = tgt_ref[p, :, 3:4]
        area_t = (tx2 - tx1) * (ty2 - ty1)
        for g in range(G):
            px1 = pred_ref[p, 0, g, :]
            py1 = pred_ref[p, 1, g, :]
            px2 = pred_ref[p, 2, g, :]
            py2 = pred_ref[p, 3, g, :]
            c_l1 = ((jnp.abs(px1 - tx1) + jnp.abs(py1 - ty1))
                    + jnp.abs(px2 - tx2)) + jnp.abs(py2 - ty2)
            area_p = (px2 - px1) * (py2 - py1)
            ltx = jnp.maximum(px1, tx1)
            lty = jnp.maximum(py1, ty1)
            rbx = jnp.minimum(px2, tx2)
            rby = jnp.minimum(py2, ty2)
            iw = jnp.maximum(rbx - ltx, 0.0)
            ih = jnp.maximum(rby - lty, 0.0)
            inter = iw * ih
            union = (area_p + area_t) - inter
            iou = inter / (union + 1e-8)
            elx = jnp.minimum(px1, tx1)
            ely = jnp.minimum(py1, ty1)
            erx = jnp.maximum(px2, tx2)
            ery = jnp.maximum(py2, ty2)
            ew = jnp.maximum(erx - elx, 0.0)
            eh = jnp.maximum(ery - ely, 0.0)
            earea = ew * eh
            giou = iou - (earea - union) / (earea + 1e-8)
            cost_ref[p, :, g, :] = c_l1 - giou

    riota = jax.lax.broadcasted_iota(jnp.int32, (1, M), 1)
    citer = jax.lax.broadcasted_iota(jnp.int32, (M, 1), 0)
    fiota = (jax.lax.broadcasted_iota(jnp.int32, (G, _LANE), 0) * _LANE
             + jax.lax.broadcasted_iota(jnp.int32, (G, _LANE), 1))

    u_ref[...] = jnp.zeros((P, M), jnp.float32)
    v_ref[...] = jnp.zeros((P, G, _LANE), jnp.float32)
    r4c_ref[...] = jnp.full((P, G, _LANE), -1, jnp.int32)
    c4r_ref[...] = jnp.full((P, M), -1, jnp.int32)
    c4rcol_ref[...] = jnp.full((P, M, 1), -1, jnp.int32)

    def outer(i, _):
        sr_ref[...] = jnp.zeros((P, M), jnp.int32)
        rowminv_ref[...] = jnp.zeros((P, M), jnp.float32)
        sc_ref[...] = jnp.zeros((P, G, _LANE), jnp.int32)
        spc_ref[...] = jnp.full((P, G, _LANE), _BIG, jnp.float32)
        path_ref[...] = jnp.zeros((P, G, _LANE), jnp.int32)
        minv_ref[...] = jnp.zeros((P, 1), jnp.float32)

        def sp_cond(s):
            m = s[P]
            for p in range(1, P):
                m = jnp.minimum(m, s[P + p])
            return m < 0

        def sp_body(s):
            new_cur, new_sink = [], []
            for p in range(P):
                cur, sink = s[p], s[P + p]
                active = sink < 0
                minv = minv_ref[p:p + 1, :]
                u = u_ref[p:p + 1, :]
                curmask = (riota == cur) & active
                sr_ref[p:p + 1, :] = jnp.where(curmask, 1, sr_ref[p:p + 1, :])
                ucur = jnp.sum(jnp.where(curmask, u, 0.0), axis=1, keepdims=True)
                c4rcur = jnp.sum(jnp.where(curmask, c4r_ref[p:p + 1, :], 0),
                                 axis=1, keepdims=True)
                pathpk = (cur + 1) * 4096 + (c4rcur + 1)
                crow = cost_ref[p, cur]
                red = ((minv + crow) - ucur) - v_ref[p]
                sc = sc_ref[p]
                spc = spc_ref[p]
                better = active & (sc == 0) & (red < spc)
                spc = jnp.where(better, red, spc)
                spc_ref[p] = spc
                path_ref[p] = jnp.where(better, pathpk, path_ref[p])
                masked = jnp.where(sc != 0, _BIG, spc)
                minv_new = jnp.min(jnp.min(masked, axis=1, keepdims=True),
                                   axis=0, keepdims=True)
                minv_ref[p:p + 1, :] = jnp.where(active, minv_new, minv)
                key = jnp.where(masked == minv_new,
                                fiota * 256 + (r4c_ref[p] + 1),
                                jnp.int32(1 << 30))
                kmin = jnp.min(key)
                j = kmin >> 8
                r4cj = (kmin & 255) - 1
                sc_ref[p] = jnp.where(active & (fiota == j), 1, sc)
                unassigned = r4cj < 0
                new_sink.append(jnp.where(active,
                                          jnp.where(unassigned, j, jnp.int32(-1)),
                                          sink))
                new_cur.append(jnp.where(active & ~unassigned, r4cj, cur))
                rowminv_ref[p:p + 1, :] = jnp.where(
                    (riota == r4cj) & active, minv_new, rowminv_ref[p:p + 1, :])
            return (*new_cur, *new_sink)

        init = tuple([i] * P) + tuple([jnp.int32(-1)] * P)
        fin = jax.lax.while_loop(sp_cond, sp_body, init)
        sinks = fin[P:]

        minv_all = minv_ref[...]
        u = u_ref[...]
        u = jnp.where(riota == i, u + minv_all, u)
        rmask = (sr_ref[...] != 0) & (riota != i)
        u_ref[...] = jnp.where(rmask, (u + minv_all) - rowminv_ref[...], u)
        for p in range(P):
            minv = minv_ref[p:p + 1, :]
            sc = sc_ref[p] != 0
            v = v_ref[p]
            v_ref[p] = jnp.where(sc, (v + spc_ref[p]) - minv, v)

        def aug_cond(s):
            m = s[P]
            for p in range(1, P):
                m = jnp.minimum(m, s[P + p])
            return m == 0

        def aug_body(s):
            new_j, new_done = [], []
            for p in range(P):
                j, done = s[p], s[P + p]
                active = done == 0
                pk = jnp.sum(jnp.where((fiota == j) & active, path_ref[p], 0))
                ii = (pk >> 12) - 1
                nj = (pk & 4095) - 1
                r4c_ref[p] = jnp.where(active & (fiota == j), ii, r4c_ref[p])
                c4r_ref[p:p + 1, :] = jnp.where(
                    (riota == ii) & active, j, c4r_ref[p:p + 1, :])
                c4rcol_ref[p] = jnp.where(
                    (citer == ii) & active, j, c4rcol_ref[p])
                new_j.append(jnp.where(active, nj, j))
                new_done.append(jnp.where(active & (ii != i), jnp.int32(0),
                                          jnp.int32(1)))
            return (*new_j, *new_done)

        jax.lax.while_loop(aug_cond, aug_body,
                           tuple(sinks) + tuple([jnp.int32(0)] * P))
        return 0

    jax.lax.fori_loop(0, M, outer, 0)

    for p in range(P):
        c4r = c4r_ref[p:p + 1, :]
        c4rcol = c4rcol_ref[p]
        less = c4r < c4rcol
        rank = jnp.sum(less.astype(jnp.int32), axis=1, keepdims=True)
        onehot = rank == riota
        pred_sorted = jnp.sum(jnp.where(onehot, c4rcol, 0), axis=0, keepdims=True)
        tgt_sorted = jnp.sum(jnp.where(onehot, citer, 0), axis=0, keepdims=True)
        out_ref[p, 0:1, :] = pred_sorted
        out_ref[p, 1:2, :] = tgt_sorted


def kernel(out_boxes, tgt_boxes):
    B, N, _ = out_boxes.shape
    M = tgt_boxes.shape[1]
    G = N // _LANE
    pred = jnp.transpose(out_boxes, (0, 2, 1)).reshape(B, 4, G, _LANE)
    pred = jnp.pad(pred, ((0, 0), (0, 4), (0, 0), (0, 0)))
    tgt = jnp.pad(tgt_boxes, ((0, 0), (0, 0), (0, 4)))
    out = pl.pallas_call(
        _matcher_body,
        grid=(B // _P,),
        in_specs=[pl.BlockSpec((_P, 8, G, _LANE), lambda b: (b, 0, 0, 0)),
                  pl.BlockSpec((_P, M, 8), lambda b: (b, 0, 0))],
        out_specs=pl.BlockSpec((_P, 8, M), lambda b: (b, 0, 0)),
        out_shape=jax.ShapeDtypeStruct((B, 8, M), jnp.int32),
        scratch_shapes=[pltpu.VMEM((_P, M, G, _LANE), jnp.float32),
                        pltpu.VMEM((_P, M), jnp.float32),
                        pltpu.VMEM((_P, G, _LANE), jnp.float32),
                        pltpu.VMEM((_P, G, _LANE), jnp.int32),
                        pltpu.VMEM((_P, M), jnp.int32),
                        pltpu.VMEM((_P, M, 1), jnp.int32),
                        pltpu.VMEM((_P, M), jnp.int32),
                        pltpu.VMEM((_P, M), jnp.float32),
                        pltpu.VMEM((_P, G, _LANE), jnp.int32),
                        pltpu.VMEM((_P, G, _LANE), jnp.float32),
                        pltpu.VMEM((_P, G, _LANE), jnp.int32),
                        pltpu.VMEM((_P, 1), jnp.float32)],
        compiler_params=pltpu.CompilerParams(
            dimension_semantics=("parallel",)),
    )(pred, tgt)
    return out[:, :2, :]
